# baseline (speedup 1.0000x reference)
import functools

import jax
import jax.numpy as jnp
from jax.experimental import pallas as pl
from jax.experimental.pallas import tpu as pltpu


def _net_kernel(x_ref, wsm_ref, bsm_ref, whead_ref, bhead_ref,
                w3_ref, b3_ref, w1_ref, b1_ref, wb_ref,
                bb_ref, wbody_ref, bbody_ref, wup_ref, bup_ref, prelu_ref,
                o_ref, *scr, H, W, nf, nb, P):
    HW = H * W
    pads = scr[:P]
    pad3s = scr[P:2 * P]
    feat_ref, cur_ref, acc_ref = scr[2 * P:]
    blk = pl.program_id(1)

    def stage(a, bufs, cin):
        for i in range(P):
            bufs[i][1:H + 1, 1:W + 1, :] = a[i * HW:(i + 1) * HW, :].reshape(H, W, cin)

    def tap(t, bufs, cin):
        kh, kw = t // 3, t % 3
        parts = [bufs[i][kh:kh + H, kw:kw + W, :].reshape(HW, cin)
                 for i in range(P)]
        return parts[0] if P == 1 else jnp.concatenate(parts, axis=0)

    @pl.when(blk == 0)
    def _():
        x = x_ref[...].reshape(P * HW, 3)
        sm = jnp.dot(x, wsm_ref[...], preferred_element_type=jnp.float32)
        sm = sm + bsm_ref[...]
        for i in range(P):
            pads[i][...] = jnp.zeros_like(pads[i])
            pad3s[i][...] = jnp.zeros_like(pad3s[i])
        stage(sm, pad3s, 3)
        h = bhead_ref[...].astype(jnp.float32)
        for t in range(9):
            h = h + jnp.dot(tap(t, pad3s, 3), whead_ref[t],
                            preferred_element_type=jnp.float32)
        feat_ref[...] = h
        cur_ref[...] = h
        acc_ref[...] = jnp.zeros_like(acc_ref)

    prelu = prelu_ref[...]
    cur = cur_ref[...]
    c3 = [0]
    cJ = [0]

    def conv3(v, act=False, residual=None):
        i = c3[0]
        c3[0] += 1
        stage(v, pads, nf)
        acc = b3_ref[0, i].astype(jnp.float32)
        for t in range(9):
            acc = acc + jnp.dot(tap(t, pads, nf), w3_ref[0, i, t * nf:(t + 1) * nf, :],
                                preferred_element_type=jnp.float32)
        if act:
            acc = jnp.where(acc >= 0.0, acc, prelu * acc)
        if residual is not None:
            acc = acc + residual
        return acc

    def conv1(chunks):
        i = cJ[0]
        cJ[0] += 1
        out = b1_ref[0, i].astype(jnp.float32)
        for k, ck in enumerate(chunks):
            out = out + jnp.dot(ck, w1_ref[0, i, k * nf:(k + 1) * nf, :],
                                preferred_element_type=jnp.float32)
        return out

    def rcb(v):
        return conv3(conv3(v, act=True), residual=v)

    def fract2(v):
        return [rcb(rcb(v)), rcb(v)]

    def fract4(v):
        ch = fract2(v)
        ch2 = fract2(conv1(ch))
        return ch2 + [rcb(v)]

    def fract8(v):
        ch = fract4(v)
        ch2 = fract4(conv1(ch))
        return ch2 + [rcb(v)]

    res = conv3(conv1(fract8(cur)), residual=cur)

    cur_ref[...] = res
    acc_ref[...] = acc_ref[...] + jnp.dot(res, wb_ref[0],
                                          preferred_element_type=jnp.float32)

    @pl.when(blk == nb - 1)
    def _():
        bottle = acc_ref[...] + bb_ref[...]
        stage(bottle, pads, nf)
        body = bbody_ref[...].astype(jnp.float32)
        for t in range(9):
            body = body + jnp.dot(tap(t, pads, nf), wbody_ref[t * nf:(t + 1) * nf, :],
                                  preferred_element_type=jnp.float32)
        body = body + feat_ref[...]
        stage(body, pads, nf)
        up = bup_ref[...].astype(jnp.float32)
        for t in range(9):
            up = up + jnp.dot(tap(t, pads, nf), wup_ref[t * nf:(t + 1) * nf, :],
                              preferred_element_type=jnp.float32)
        o_ref[...] = up.reshape(P, H, W, 4 * nf).astype(o_ref.dtype)


def _build_net(N, H, W, nf, nb, P):
    body = functools.partial(_net_kernel, H=H, W=W, nf=nf, nb=nb, P=P)
    scratch = [pltpu.VMEM((H + 2, W + 2, nf), jnp.float32) for _ in range(P)]
    scratch += [pltpu.VMEM((H + 2, W + 2, 3), jnp.float32) for _ in range(P)]
    scratch += [pltpu.VMEM((P * H * W, nf), jnp.float32) for _ in range(3)]
    return pl.pallas_call(
        body,
        out_shape=jax.ShapeDtypeStruct((N, H, W, 4 * nf), jnp.float32),
        grid=(N // P, nb),
        in_specs=[
            pl.BlockSpec((P, H, W, 3), lambda n, i: (n, 0, 0, 0)),
            pl.BlockSpec((3, 3), lambda n, i: (0, 0)),
            pl.BlockSpec((1, 3), lambda n, i: (0, 0)),
            pl.BlockSpec((9, 3, nf), lambda n, i: (0, 0, 0)),
            pl.BlockSpec((1, nf), lambda n, i: (0, 0)),
            pl.BlockSpec((1, 31, 9 * nf, nf), lambda n, i: (i, 0, 0, 0)),
            pl.BlockSpec((1, 31, 1, nf), lambda n, i: (i, 0, 0, 0)),
            pl.BlockSpec((1, 4, 4 * nf, nf), lambda n, i: (i, 0, 0, 0)),
            pl.BlockSpec((1, 4, 1, nf), lambda n, i: (i, 0, 0, 0)),
            pl.BlockSpec((1, nf, nf), lambda n, i: (i, 0, 0)),
            pl.BlockSpec((1, nf), lambda n, i: (0, 0)),
            pl.BlockSpec((9 * nf, nf), lambda n, i: (0, 0)),
            pl.BlockSpec((1, nf), lambda n, i: (0, 0)),
            pl.BlockSpec((9 * nf, 4 * nf), lambda n, i: (0, 0)),
            pl.BlockSpec((1, 4 * nf), lambda n, i: (0, 0)),
            pl.BlockSpec((1, 1), lambda n, i: (0, 0)),
        ],
        out_specs=pl.BlockSpec((P, H, W, 4 * nf), lambda n, i: (n, 0, 0, 0)),
        scratch_shapes=scratch,
        compiler_params=pltpu.CompilerParams(
            dimension_semantics=("parallel", "arbitrary")),
    )


def _tail_kernel(t_ref, wt_ref, bt_ref, wam_ref, bam_ref, o_ref, pad_ref,
                 *, H, W, nf):
    @pl.when(pl.program_id(0) == 0)
    def _():
        pad_ref[...] = jnp.zeros_like(pad_ref)

    pad_ref[1:H + 1, 1:W + 1, :] = t_ref[0]
    y = bt_ref[...].astype(jnp.float32)
    for t in range(9):
        kh, kw = t // 3, t % 3
        patch = pad_ref[kh:kh + H, kw:kw + W, :].reshape(H * W, nf)
        y = y + jnp.dot(patch, wt_ref[t * nf:(t + 1) * nf, :],
                        preferred_element_type=jnp.float32)
    y = jnp.dot(y, wam_ref[...], preferred_element_type=jnp.float32) + bam_ref[...]
    o_ref[0] = y.reshape(H, W, 3).astype(o_ref.dtype)


def _build_tail(N, H, W, nf):
    body = functools.partial(_tail_kernel, H=H, W=W, nf=nf)
    return pl.pallas_call(
        body,
        out_shape=jax.ShapeDtypeStruct((N, H, W, 3), jnp.float32),
        grid=(N,),
        in_specs=[
            pl.BlockSpec((1, H, W, nf), lambda n: (n, 0, 0, 0)),
            pl.BlockSpec((9 * nf, 3), lambda n: (0, 0)),
            pl.BlockSpec((1, 3), lambda n: (0, 0)),
            pl.BlockSpec((3, 3), lambda n: (0, 0)),
            pl.BlockSpec((1, 3), lambda n: (0, 0)),
        ],
        out_specs=pl.BlockSpec((1, H, W, 3), lambda n: (n, 0, 0, 0)),
        scratch_shapes=[pltpu.VMEM((H + 2, W + 2, nf), jnp.float32)],
        compiler_params=pltpu.CompilerParams(dimension_semantics=("parallel",)),
    )


def _pixel_shuffle(x, r):
    N, H, W, C = x.shape
    c = C // (r * r)
    x = x.reshape(N, H, W, c, r, r)
    x = jnp.transpose(x, (0, 1, 4, 2, 5, 3))
    return x.reshape(N, H * r, W * r, c)


def kernel(x, w_sm, b_sm, w_am, b_am, w_head, b_head, w3, b3, w1, b1, wb, bb,
           w_body, b_body, w_up, b_up, w_tail, b_tail, prelu):
    N, H, W, _ = x.shape
    nf = b_head.shape[1]
    nb = w3.shape[0]
    scale = 2
    P = 1

    w3_p = w3.reshape(nb, 31, 9 * nf, nf)
    wbody_p = w_body.reshape(9 * nf, nf)
    wup_p = w_up.reshape(9 * nf, 4 * nf)
    wt_p = w_tail.reshape(9 * nf, 3)

    up = _build_net(N, H, W, nf, nb, P)(
        x, w_sm, b_sm, w_head, b_head, w3_p, b3, w1, b1, wb, bb,
        wbody_p, b_body, wup_p, b_up, prelu)

    t = _pixel_shuffle(up, scale)
    return _build_tail(N, H * scale, W * scale, nf)(t, wt_p, b_tail, w_am, b_am)


# speedup vs baseline: 1.0068x; 1.0068x over previous
import functools

import jax
import jax.numpy as jnp
from jax.experimental import pallas as pl
from jax.experimental.pallas import tpu as pltpu


def _net_kernel(x_ref, wsm_ref, bsm_ref, whead_ref, bhead_ref,
                w3_ref, b3_ref, w1_ref, b1_ref, wb_ref,
                bb_ref, wbody_ref, bbody_ref, wup_ref, bup_ref, prelu_ref,
                o_ref, *scr, H, W, nf, nb, P):
    HW = H * W
    pads = scr[:P]
    pad3s = scr[P:2 * P]
    feat_ref, cur_ref, acc_ref = scr[2 * P:]
    blk = pl.program_id(1)
    for i in range(P):
        pads[i][...] = jnp.zeros_like(pads[i])

    def stage(a, bufs, cin):
        for i in range(P):
            bufs[i][1:H + 1, 1:W + 1, :] = a[i * HW:(i + 1) * HW, :].reshape(H, W, cin)

    def tap(t, bufs, cin):
        kh, kw = t // 3, t % 3
        parts = [bufs[i][kh:kh + H, kw:kw + W, :].reshape(HW, cin)
                 for i in range(P)]
        return parts[0] if P == 1 else jnp.concatenate(parts, axis=0)

    @pl.when(blk == 0)
    def _():
        x = x_ref[...].reshape(P * HW, 3)
        sm = jnp.dot(x, wsm_ref[...], preferred_element_type=jnp.float32)
        sm = sm + bsm_ref[...]
        for i in range(P):
            pad3s[i][...] = jnp.zeros_like(pad3s[i])
        stage(sm, pad3s, 3)
        h = bhead_ref[...].astype(jnp.float32)
        for t in range(9):
            h = h + jnp.dot(tap(t, pad3s, 3), whead_ref[t],
                            preferred_element_type=jnp.float32)
        feat_ref[...] = h
        cur_ref[...] = h
        acc_ref[...] = jnp.zeros_like(acc_ref)

    prelu = prelu_ref[...]
    cur = cur_ref[...]
    c3 = [0]
    cJ = [0]

    def conv3(v, act=False, residual=None):
        i = c3[0]
        c3[0] += 1
        stage(v, pads, nf)
        acc = b3_ref[0, i].astype(jnp.float32)
        for t in range(9):
            acc = acc + jnp.dot(tap(t, pads, nf), w3_ref[0, i, t * nf:(t + 1) * nf, :],
                                preferred_element_type=jnp.float32)
        if act:
            acc = jnp.where(acc >= 0.0, acc, prelu * acc)
        if residual is not None:
            acc = acc + residual
        return acc

    def conv1(chunks):
        i = cJ[0]
        cJ[0] += 1
        out = b1_ref[0, i].astype(jnp.float32)
        for k, ck in enumerate(chunks):
            out = out + jnp.dot(ck, w1_ref[0, i, k * nf:(k + 1) * nf, :],
                                preferred_element_type=jnp.float32)
        return out

    def rcb(v):
        return conv3(conv3(v, act=True), residual=v)

    def fract2(v):
        return [rcb(rcb(v)), rcb(v)]

    def fract4(v):
        ch = fract2(v)
        ch2 = fract2(conv1(ch))
        return ch2 + [rcb(v)]

    def fract8(v):
        ch = fract4(v)
        ch2 = fract4(conv1(ch))
        return ch2 + [rcb(v)]

    res = conv3(conv1(fract8(cur)), residual=cur)

    cur_ref[...] = res
    acc_ref[...] = acc_ref[...] + jnp.dot(res, wb_ref[0],
                                          preferred_element_type=jnp.float32)

    @pl.when(blk == nb - 1)
    def _():
        bottle = acc_ref[...] + bb_ref[...]
        stage(bottle, pads, nf)
        body = bbody_ref[...].astype(jnp.float32)
        for t in range(9):
            body = body + jnp.dot(tap(t, pads, nf), wbody_ref[t * nf:(t + 1) * nf, :],
                                  preferred_element_type=jnp.float32)
        body = body + feat_ref[...]
        stage(body, pads, nf)
        up = bup_ref[...].astype(jnp.float32)
        for t in range(9):
            up = up + jnp.dot(tap(t, pads, nf), wup_ref[t * nf:(t + 1) * nf, :],
                              preferred_element_type=jnp.float32)
        o_ref[...] = up.reshape(P, H, W, 4 * nf).astype(o_ref.dtype)


def _build_net(N, H, W, nf, nb, P):
    body = functools.partial(_net_kernel, H=H, W=W, nf=nf, nb=nb, P=P)
    scratch = [pltpu.VMEM((H + 2, W + 2, nf), jnp.float32) for _ in range(P)]
    scratch += [pltpu.VMEM((H + 2, W + 2, 3), jnp.float32) for _ in range(P)]
    scratch += [pltpu.VMEM((P * H * W, nf), jnp.float32) for _ in range(3)]
    return pl.pallas_call(
        body,
        out_shape=jax.ShapeDtypeStruct((N, H, W, 4 * nf), jnp.float32),
        grid=(N // P, nb),
        in_specs=[
            pl.BlockSpec((P, H, W, 3), lambda n, i: (n, 0, 0, 0)),
            pl.BlockSpec((3, 3), lambda n, i: (0, 0)),
            pl.BlockSpec((1, 3), lambda n, i: (0, 0)),
            pl.BlockSpec((9, 3, nf), lambda n, i: (0, 0, 0)),
            pl.BlockSpec((1, nf), lambda n, i: (0, 0)),
            pl.BlockSpec((1, 31, 9 * nf, nf), lambda n, i: (i, 0, 0, 0)),
            pl.BlockSpec((1, 31, 1, nf), lambda n, i: (i, 0, 0, 0)),
            pl.BlockSpec((1, 4, 4 * nf, nf), lambda n, i: (i, 0, 0, 0)),
            pl.BlockSpec((1, 4, 1, nf), lambda n, i: (i, 0, 0, 0)),
            pl.BlockSpec((1, nf, nf), lambda n, i: (i, 0, 0)),
            pl.BlockSpec((1, nf), lambda n, i: (0, 0)),
            pl.BlockSpec((9 * nf, nf), lambda n, i: (0, 0)),
            pl.BlockSpec((1, nf), lambda n, i: (0, 0)),
            pl.BlockSpec((9 * nf, 4 * nf), lambda n, i: (0, 0)),
            pl.BlockSpec((1, 4 * nf), lambda n, i: (0, 0)),
            pl.BlockSpec((1, 1), lambda n, i: (0, 0)),
        ],
        out_specs=pl.BlockSpec((P, H, W, 4 * nf), lambda n, i: (n, 0, 0, 0)),
        scratch_shapes=scratch,
        compiler_params=pltpu.CompilerParams(
            dimension_semantics=("parallel", "arbitrary")),
    )


def _tail_kernel(t_ref, wt_ref, bt_ref, wam_ref, bam_ref, o_ref, pad_ref,
                 *, H, W, nf):
    @pl.when(pl.program_id(0) == 0)
    def _():
        pad_ref[...] = jnp.zeros_like(pad_ref)

    pad_ref[1:H + 1, 1:W + 1, :] = t_ref[0]
    y = bt_ref[...].astype(jnp.float32)
    for t in range(9):
        kh, kw = t // 3, t % 3
        patch = pad_ref[kh:kh + H, kw:kw + W, :].reshape(H * W, nf)
        y = y + jnp.dot(patch, wt_ref[t * nf:(t + 1) * nf, :],
                        preferred_element_type=jnp.float32)
    y = jnp.dot(y, wam_ref[...], preferred_element_type=jnp.float32) + bam_ref[...]
    o_ref[0] = y.reshape(H, W, 3).astype(o_ref.dtype)


def _build_tail(N, H, W, nf):
    body = functools.partial(_tail_kernel, H=H, W=W, nf=nf)
    return pl.pallas_call(
        body,
        out_shape=jax.ShapeDtypeStruct((N, H, W, 3), jnp.float32),
        grid=(N,),
        in_specs=[
            pl.BlockSpec((1, H, W, nf), lambda n: (n, 0, 0, 0)),
            pl.BlockSpec((9 * nf, 3), lambda n: (0, 0)),
            pl.BlockSpec((1, 3), lambda n: (0, 0)),
            pl.BlockSpec((3, 3), lambda n: (0, 0)),
            pl.BlockSpec((1, 3), lambda n: (0, 0)),
        ],
        out_specs=pl.BlockSpec((1, H, W, 3), lambda n: (n, 0, 0, 0)),
        scratch_shapes=[pltpu.VMEM((H + 2, W + 2, nf), jnp.float32)],
        compiler_params=pltpu.CompilerParams(dimension_semantics=("parallel",)),
    )


def _pixel_shuffle(x, r):
    N, H, W, C = x.shape
    c = C // (r * r)
    x = x.reshape(N, H, W, c, r, r)
    x = jnp.transpose(x, (0, 1, 4, 2, 5, 3))
    return x.reshape(N, H * r, W * r, c)


def kernel(x, w_sm, b_sm, w_am, b_am, w_head, b_head, w3, b3, w1, b1, wb, bb,
           w_body, b_body, w_up, b_up, w_tail, b_tail, prelu):
    N, H, W, _ = x.shape
    nf = b_head.shape[1]
    nb = w3.shape[0]
    scale = 2
    P = 1

    w3_p = w3.reshape(nb, 31, 9 * nf, nf)
    wbody_p = w_body.reshape(9 * nf, nf)
    wup_p = w_up.reshape(9 * nf, 4 * nf)
    wt_p = w_tail.reshape(9 * nf, 3)

    up = _build_net(N, H, W, nf, nb, P)(
        x, w_sm, b_sm, w_head, b_head, w3_p, b3, w1, b1, wb, bb,
        wbody_p, b_body, wup_p, b_up, prelu)

    t = _pixel_shuffle(up, scale)
    return _build_tail(N, H * scale, W * scale, nf)(t, wt_p, b_tail, w_am, b_am)
